# baseline (speedup 1.0000x reference)
import functools

import jax
import jax.numpy as jnp
from jax import lax
from jax.experimental import pallas as pl
from jax.experimental.pallas import tpu as pltpu
from jax.experimental.pallas import tpu_sc as plsc


def _prep_body(t_ref, q_ref, lo_ref, w_ref):
    t_col = t_ref[...]
    q_row = q_ref[...]
    mask = t_col <= q_row
    cnt = jnp.sum(mask.astype(jnp.int32), axis=0, keepdims=True)
    tmax = t_col[-1:, :]
    tmin = t_col[:1, :]
    t0 = jnp.max(jnp.where(mask, t_col, tmin - 1.0), axis=0, keepdims=True)
    t1 = jnp.min(jnp.where(mask, tmax + 1.0, t_col), axis=0, keepdims=True)
    idx = jnp.clip(cnt, 1, t_ref.shape[0] - 1)
    lo_ref[...] = idx - 1
    w_ref[...] = (q_row - t0) / (t1 - t0)


def _prep(query_t, t):
    T = t.shape[0]
    S = query_t.shape[0]
    lo, w = pl.pallas_call(
        _prep_body,
        out_shape=(
            jax.ShapeDtypeStruct((1, S), jnp.int32),
            jax.ShapeDtypeStruct((1, S), jnp.float32),
        ),
    )(t.reshape(T, 1), query_t.reshape(1, S))
    return lo.reshape(S), w.reshape(S)


def _sc_spline(lo, w, x0, knots, x1, *, B, T, D, S):
    info = plsc.get_sparse_core_info()
    NC, NS = info.num_cores, info.num_subcores
    NW = NC * NS
    assert B % NW == 0
    nb = B // NW

    mesh = plsc.VectorSubcoreMesh(core_axis_name="c", subcore_axis_name="s")

    @functools.partial(
        pl.kernel,
        out_type=jax.ShapeDtypeStruct((B, S, D), jnp.float32),
        mesh=mesh,
        scratch_types=[
            pltpu.VMEM((T, D), jnp.float32),
            pltpu.VMEM((S, D), jnp.float32),
            pltpu.VMEM((S,), jnp.int32),
            pltpu.VMEM((S,), jnp.float32),
        ],
        compiler_params=pltpu.CompilerParams(
            use_tc_tiling_on_sc=False,
            needs_layout_passes=False,
        ),
    )
    def run(lo_hbm, w_hbm, x0_hbm, knots_hbm, x1_hbm, out_hbm, col, outb, lo_v, w_v):
        wid = lax.axis_index("s") * NC + lax.axis_index("c")
        pltpu.sync_copy(lo_hbm, lo_v)
        pltpu.sync_copy(w_hbm, w_v)

        def col_loop(j, carry):
            b = wid * nb + j
            pltpu.sync_copy(x0_hbm.at[0, b], col.at[0])
            pltpu.sync_copy(knots_hbm.at[:, b], col.at[pl.ds(1, T - 2)])
            pltpu.sync_copy(x1_hbm.at[0, b], col.at[T - 1])

            def g_loop(g, carry2):
                s0 = g * 16
                lo16 = lo_v[pl.ds(s0, 16)]
                hi16 = lo16 + 1
                w16 = w_v[pl.ds(s0, 16)]
                q16 = lax.iota(jnp.int32, 16) + s0

                def d_loop(d, carry3):
                    d16 = jnp.zeros((16,), jnp.int32) + d
                    glo = plsc.load_gather(col, [lo16, d16])
                    ghi = plsc.load_gather(col, [hi16, d16])
                    plsc.store_scatter(outb, [q16, d16], glo + w16 * (ghi - glo))
                    return carry3

                lax.fori_loop(0, D, d_loop, 0)
                return carry2

            lax.fori_loop(0, S // 16, g_loop, 0)
            pltpu.sync_copy(outb, out_hbm.at[b])
            return carry

        lax.fori_loop(0, nb, col_loop, 0)

    return run(lo, w, x0, knots, x1)


def kernel(query_t, t, x0, knots, x1):
    T = t.shape[0]
    S = query_t.shape[0]
    _, B, D = knots.shape[0] + 2, knots.shape[1], knots.shape[2]
    lo, w = _prep(query_t, t)
    return _sc_spline(lo, w, x0, knots, x1, B=B, T=T, D=D, S=S)


# speedup vs baseline: 2.9033x; 2.9033x over previous
import functools

import jax
import jax.numpy as jnp
from jax import lax
from jax.experimental import pallas as pl
from jax.experimental.pallas import tpu as pltpu
from jax.experimental.pallas import tpu_sc as plsc


def _prep_body(t_ref, q_ref, lo_ref, hi_ref, wb_ref):
    t_col = t_ref[...]
    q_row = q_ref[...]
    mask = t_col <= q_row
    cnt = jnp.sum(mask.astype(jnp.int32), axis=0, keepdims=True)
    tmax = t_col[-1:, :]
    tmin = t_col[:1, :]
    t0 = jnp.max(jnp.where(mask, t_col, tmin - 1.0), axis=0, keepdims=True)
    t1 = jnp.min(jnp.where(mask, tmax + 1.0, t_col), axis=0, keepdims=True)
    idx = jnp.clip(cnt, 1, t_ref.shape[0] - 1)
    lo_ref[...] = idx - 1
    hi_ref[...] = idx
    w = (q_row - t0) / (t1 - t0)
    wb_ref[...] = jnp.broadcast_to(w.T, wb_ref.shape)


def _prep(query_t, t, D):
    T = t.shape[0]
    S = query_t.shape[0]
    lo, hi, wb = pl.pallas_call(
        _prep_body,
        out_shape=(
            jax.ShapeDtypeStruct((1, S), jnp.int32),
            jax.ShapeDtypeStruct((1, S), jnp.int32),
            jax.ShapeDtypeStruct((S, D), jnp.float32),
        ),
    )(t.reshape(T, 1), query_t.reshape(1, S))
    return lo.reshape(S), hi.reshape(S), wb


def _sc_spline(lo, hi, wb, x0, knots, x1, *, B, T, D, S):
    info = plsc.get_sparse_core_info()
    NC, NS = info.num_cores, info.num_subcores
    NW = NC * NS
    assert B % NW == 0
    nb = B // NW
    SH = S // 4

    mesh = plsc.VectorSubcoreMesh(core_axis_name="c", subcore_axis_name="s")

    @functools.partial(
        pl.kernel,
        out_type=jax.ShapeDtypeStruct((B, S, D), jnp.float32),
        mesh=mesh,
        scratch_types=[
            pltpu.VMEM_SHARED((NS, T, D), jnp.float32),
            pltpu.VMEM((SH, D), jnp.float32),
            pltpu.VMEM((SH, D), jnp.float32),
            pltpu.VMEM((S, D), jnp.float32),
            pltpu.VMEM((S,), jnp.int32),
            pltpu.VMEM((S,), jnp.int32),
        ],
        compiler_params=pltpu.CompilerParams(
            use_tc_tiling_on_sc=False,
            needs_layout_passes=False,
        ),
    )
    def run(lo_hbm, hi_hbm, wb_hbm, x0_hbm, knots_hbm, x1_hbm, out_hbm,
            col_sh, glo, ghi, w_b, lo_v, hi_v):
        sid = lax.axis_index("s")
        wid = sid * NC + lax.axis_index("c")
        col = col_sh.at[sid]
        pltpu.sync_copy(lo_hbm, lo_v)
        pltpu.sync_copy(hi_hbm, hi_v)
        pltpu.sync_copy(wb_hbm, w_b)

        def col_loop(j, carry):
            b = wid * nb + j
            pltpu.sync_copy(x0_hbm.at[0, b], col.at[0])
            pltpu.sync_copy(knots_hbm.at[:, b], col.at[pl.ds(1, T - 2)])
            pltpu.sync_copy(x1_hbm.at[0, b], col.at[T - 1])

            for h in range(S // SH):
                pltpu.sync_copy(col.at[lo_v.at[pl.ds(h * SH, SH)]], glo)
                pltpu.sync_copy(col.at[hi_v.at[pl.ds(h * SH, SH)]], ghi)

                def s_loop(s, carry2):
                    sg = h * SH + s
                    for dc in range(D // 16):
                        dsl = pl.ds(dc * 16, 16)
                        a = glo[s, dsl]
                        c = ghi[s, dsl]
                        w16 = w_b[sg, dsl]
                        glo[s, dsl] = a + w16 * (c - a)
                    return carry2

                lax.fori_loop(0, SH, s_loop, 0)
                pltpu.sync_copy(glo, out_hbm.at[b, pl.ds(h * SH, SH)])
            return carry

        lax.fori_loop(0, nb, col_loop, 0)

    return run(lo, hi, wb, x0, knots, x1)


def kernel(query_t, t, x0, knots, x1):
    T = t.shape[0]
    S = query_t.shape[0]
    B, D = knots.shape[1], knots.shape[2]
    lo, hi, wb = _prep(query_t, t, D)
    return _sc_spline(lo, hi, wb, x0, knots, x1, B=B, T=T, D=D, S=S)
